# baseline (speedup 1.0000x reference)
import functools

import jax
import jax.numpy as jnp
from jax import lax
from jax.experimental import pallas as pl
from jax.experimental.pallas import tpu as pltpu
from jax.experimental.pallas import tpu_sc as plsc

B, T = 4096, 200
D = 2
N = B * T
V = 1000000
VPAD = 1000448
ROWS = 250112
PLANE_ROWS = 125056

_info = plsc.get_sparse_core_info()
NC, NS, L = _info.num_cores, _info.num_subcores, _info.num_lanes
NW = NC * NS
PER_W = N // NW
C = 640
NCH = PER_W // C
DEPTH = 4
GR = C // L
UNROLL = 4


def _gather_body(idx_hbm, table_hbm, out_hbm, idx_v, out_v, *bufs):
    sx_b = bufs[0:DEPTH]
    sy_b = bufs[DEPTH:2 * DEPTH]
    col_b = bufs[2 * DEPTH:3 * DEPTH]
    rx_b = bufs[3 * DEPTH:4 * DEPTH]
    ry_b = bufs[4 * DEPTH:5 * DEPTH]
    semx = bufs[5 * DEPTH:6 * DEPTH]
    semy = bufs[6 * DEPTH:7 * DEPTH]

    wid = lax.axis_index("s") * NC + lax.axis_index("c")
    pltpu.sync_copy(idx_hbm.at[:, wid], idx_v)
    lane = lax.iota(jnp.int32, L)

    def prep_group(m, d, j):
        t = m // 128
        c0 = m - t * 128
        v = idx_v[t // 8, pl.ds((t % 8) * 128 + c0, L)]
        sx = lax.shift_right_logical(v, 3)
        sx_b[d][pl.ds(j, L)] = sx
        sy_b[d][pl.ds(j, L)] = sx + PLANE_ROWS
        col_b[d][pl.ds(j, L)] = v & 7

    def sel_group(m, d, j):
        rl = lane + j
        cf = col_b[d][pl.ds(j, L)]
        v0 = plsc.load_gather(rx_b[d], [rl, cf])
        v1 = plsc.load_gather(ry_b[d], [rl, cf])
        tq = m // 128
        cs = m - tq * 128
        out_v[tq, pl.ds(cs, L)] = v0
        out_v[tq, pl.ds(cs + 128, L)] = v1

    def prep(c, d):
        def step(i, carry):
            j0 = i * (L * UNROLL)
            for u in range(UNROLL):
                j = j0 + u * L
                prep_group(c * C + j, d, j)
            return carry
        lax.fori_loop(0, GR // UNROLL, step, 0)

    def select(c, d):
        def step(i, carry):
            j0 = i * (L * UNROLL)
            for u in range(UNROLL):
                j = j0 + u * L
                sel_group(c * C + j, d, j)
            return carry
        lax.fori_loop(0, GR // UNROLL, step, 0)

    def fire(d):
        pltpu.async_copy(table_hbm.at[sx_b[d]], rx_b[d], semx[d])
        pltpu.async_copy(table_hbm.at[sy_b[d]], ry_b[d], semy[d])

    def drain(d):
        pltpu.make_async_copy(table_hbm.at[sx_b[d]], rx_b[d], semx[d]).wait()
        pltpu.make_async_copy(table_hbm.at[sy_b[d]], ry_b[d], semy[d]).wait()

    for d in range(DEPTH - 1):
        prep(d, d)
        fire(d)

    def pipe(i, carry):
        cbase = i * DEPTH
        for d in range(DEPTH):
            c = cbase + d
            cn = c + DEPTH - 1
            dn = (d + DEPTH - 1) % DEPTH

            @pl.when(cn < NCH)
            def _():
                prep(cn, dn)
                fire(dn)

            drain(d)
            select(c, d)
        return carry

    lax.fori_loop(0, NCH // DEPTH, pipe, 0)
    pltpu.sync_copy(out_v, out_hbm.at[:, wid])


@jax.jit
def _gather(ids_native, table8):
    mesh = plsc.VectorSubcoreMesh(core_axis_name="c", subcore_axis_name="s")
    run = pl.kernel(
        _gather_body,
        out_type=jax.ShapeDtypeStruct((T, NW, 2 * 128), jnp.float32),
        mesh=mesh,
        scratch_types=(
            [pltpu.VMEM((25, 1024), jnp.int32),
             pltpu.VMEM((T, 2 * 128), jnp.float32)]
            + [pltpu.VMEM((C,), jnp.int32) for _ in range(3 * DEPTH)]
            + [pltpu.VMEM((C, 8), jnp.float32) for _ in range(2 * DEPTH)]
            + [pltpu.SemaphoreType.DMA for _ in range(2 * DEPTH)]
        ),
        compiler_params=pltpu.CompilerParams(
            use_tc_tiling_on_sc=False, needs_layout_passes=False
        ),
    )
    return run(ids_native, table8)


def kernel(sensor_ids, positions):
    ids = sensor_ids.astype(jnp.int32)
    ids_native = (
        ids.reshape(32, 128, 25, 8)
        .transpose(2, 0, 3, 1)
        .reshape(25, 32, 1024)
    )
    table8 = (
        jnp.pad(positions, ((0, VPAD - V), (0, 0)))
        .reshape(VPAD // 128, 128, 2)
        .transpose(2, 0, 1)
        .reshape(ROWS, 8)
    )
    out3 = _gather(ids_native, table8)
    return (
        out3.reshape(T, NW, 2, 128)
        .transpose(1, 3, 0, 2)
        .reshape(B, T, D)
    )


# speedup vs baseline: 1.1330x; 1.1330x over previous
import functools

import jax
import jax.numpy as jnp
from jax import lax
from jax.experimental import pallas as pl
from jax.experimental.pallas import tpu as pltpu
from jax.experimental.pallas import tpu_sc as plsc

B, T = 4096, 200
D = 2
N = B * T
V = 1000000
VPAD = 1000448
ROWS = 250112
PLANE_ROWS = 125056

_info = plsc.get_sparse_core_info()
NC, NS, L = _info.num_cores, _info.num_subcores, _info.num_lanes
NW = NC * NS
PER_W = N // NW
C = 640
NCH = PER_W // C
DEPTH = 4
GR = C // L
UNROLL = 4


def _gather_body(idx_hbm, table_hbm, out_hbm, idx_v, out_v, *bufs):
    sx_b = bufs[0:DEPTH]
    sy_b = bufs[DEPTH:2 * DEPTH]
    col_b = bufs[2 * DEPTH:3 * DEPTH]
    rx_b = bufs[3 * DEPTH:4 * DEPTH]
    ry_b = bufs[4 * DEPTH:5 * DEPTH]
    semx = bufs[5 * DEPTH:6 * DEPTH]
    semy = bufs[6 * DEPTH:7 * DEPTH]

    wid = lax.axis_index("s") * NC + lax.axis_index("c")
    pltpu.sync_copy(idx_hbm.at[:, wid], idx_v)
    lane = lax.iota(jnp.int32, L)

    def prep_group(m, d, j):
        t = m // 128
        c0 = m - t * 128
        v = idx_v[t // 8, pl.ds((t % 8) * 128 + c0, L)]
        sx = lax.shift_right_logical(v, 3) + lax.shift_left(
            lax.shift_right_logical(v, 7), 4)
        sx_b[d][pl.ds(j, L)] = sx
        sy_b[d][pl.ds(j, L)] = sx + 16
        col_b[d][pl.ds(j, L)] = v & 7

    def sel_group(m, d, j):
        rl = lane + j
        cf = col_b[d][pl.ds(j, L)]
        v0 = plsc.load_gather(rx_b[d], [rl, cf])
        v1 = plsc.load_gather(ry_b[d], [rl, cf])
        tq = m // 128
        cs = m - tq * 128
        out_v[tq, pl.ds(cs, L)] = v0
        out_v[tq, pl.ds(cs + 128, L)] = v1

    def prep(c, d):
        def step(i, carry):
            j0 = i * (L * UNROLL)
            for u in range(UNROLL):
                j = j0 + u * L
                prep_group(c * C + j, d, j)
            return carry
        lax.fori_loop(0, GR // UNROLL, step, 0)

    def select(c, d):
        def step(i, carry):
            j0 = i * (L * UNROLL)
            for u in range(UNROLL):
                j = j0 + u * L
                sel_group(c * C + j, d, j)
            return carry
        lax.fori_loop(0, GR // UNROLL, step, 0)

    def fire(d):
        pltpu.async_copy(table_hbm.at[sx_b[d]], rx_b[d], semx[d])
        pltpu.async_copy(table_hbm.at[sy_b[d]], ry_b[d], semy[d])

    def drain(d):
        pltpu.make_async_copy(table_hbm.at[sx_b[d]], rx_b[d], semx[d]).wait()
        pltpu.make_async_copy(table_hbm.at[sy_b[d]], ry_b[d], semy[d]).wait()

    for d in range(DEPTH - 1):
        prep(d, d)
        fire(d)

    def pipe(i, carry):
        cbase = i * DEPTH
        for d in range(DEPTH):
            c = cbase + d
            cn = c + DEPTH - 1
            dn = (d + DEPTH - 1) % DEPTH

            @pl.when(cn < NCH)
            def _():
                prep(cn, dn)
                fire(dn)

            drain(d)
            select(c, d)
        return carry

    lax.fori_loop(0, NCH // DEPTH, pipe, 0)
    pltpu.sync_copy(out_v, out_hbm.at[:, wid])


@jax.jit
def _gather(ids_native, table8):
    mesh = plsc.VectorSubcoreMesh(core_axis_name="c", subcore_axis_name="s")
    run = pl.kernel(
        _gather_body,
        out_type=jax.ShapeDtypeStruct((T, NW, 2 * 128), jnp.float32),
        mesh=mesh,
        scratch_types=(
            [pltpu.VMEM((25, 1024), jnp.int32),
             pltpu.VMEM((T, 2 * 128), jnp.float32)]
            + [pltpu.VMEM((C,), jnp.int32) for _ in range(3 * DEPTH)]
            + [pltpu.VMEM((C, 8), jnp.float32) for _ in range(2 * DEPTH)]
            + [pltpu.SemaphoreType.DMA for _ in range(2 * DEPTH)]
        ),
        compiler_params=pltpu.CompilerParams(
            use_tc_tiling_on_sc=False, needs_layout_passes=False
        ),
    )
    return run(ids_native, table8)


def kernel(sensor_ids, positions):
    ids = sensor_ids.astype(jnp.int32)
    ids_native = (
        ids.reshape(32, 128, 25, 8)
        .transpose(2, 0, 3, 1)
        .reshape(25, 32, 1024)
    )
    table8 = (
        jnp.pad(positions, ((0, VPAD - V), (0, 0)))
        .reshape(1, VPAD // 128, 128, 2)
        .transpose(0, 1, 3, 2)
        .reshape(ROWS, 8)
    )
    out3 = _gather(ids_native, table8)
    return (
        out3.reshape(T, NW, 2, 128)
        .transpose(1, 3, 0, 2)
        .reshape(B, T, D)
    )
